# baseline (speedup 1.0000x reference)
import functools

import jax
import jax.numpy as jnp
from jax import lax
from jax.experimental import pallas as pl
from jax.experimental.pallas import tpu as pltpu
from jax.experimental.pallas import tpu_sc as plsc

_EPS = 1e-12
_NC = 2
_NS = 16
_NW = _NC * _NS
_CHUNK = 128


def _sc_gather(table, idx):
    n = idx.shape[0]
    d = table.shape[1]
    n_chunks = n // (_NW * _CHUNK)
    idx3 = idx.reshape(_NW, n_chunks, _CHUNK)

    mesh = plsc.VectorSubcoreMesh(
        core_axis_name="c", subcore_axis_name="s",
        num_cores=_NC, num_subcores=_NS)

    @functools.partial(
        pl.kernel,
        out_type=jax.ShapeDtypeStruct((n, d), jnp.float32),
        mesh=mesh,
        scratch_types=[
            pltpu.VMEM((n_chunks, _CHUNK), jnp.int32),
            pltpu.VMEM((_CHUNK, d), jnp.float32),
            pltpu.VMEM((_CHUNK, d), jnp.float32),
            pltpu.SemaphoreType.DMA,
            pltpu.SemaphoreType.DMA,
        ],
        compiler_params=pltpu.CompilerParams(use_tc_tiling_on_sc=True),
    )
    def gather_kernel(table_hbm, idx_hbm, out_hbm, idx_v, buf0, buf1, sem0, sem1):
        wid = lax.axis_index("s") * _NC + lax.axis_index("c")
        pltpu.sync_copy(idx_hbm.at[wid], idx_v)
        bufs = (buf0, buf1)
        sems = (sem0, sem1)
        base = wid * (n_chunks * _CHUNK)
        cps = [None] * n_chunks
        cps[0] = pltpu.async_copy(table_hbm.at[idx_v.at[0]], bufs[0], sems[0])
        for j in range(n_chunks):
            if j + 1 < n_chunks:
                cps[j + 1] = pltpu.async_copy(
                    table_hbm.at[idx_v.at[j + 1]], bufs[(j + 1) % 2], sems[(j + 1) % 2])
            cps[j].wait()
            pltpu.sync_copy(bufs[j % 2], out_hbm.at[pl.ds(base + j * _CHUNK, _CHUNK)])

    return gather_kernel(table, idx3)


def _tc_body(maxpos, m, r, bb, l, ent_ref, pid_ref, pos_ref, w_ref, g_ref, b_ref,
             out_ref):
    proj = jnp.dot(ent_ref[...], w_ref[...], preferred_element_type=jnp.float32)
    pid = pid_ref[...]
    iota = lax.broadcasted_iota(jnp.int32, (r, maxpos), 1)
    counts = jnp.zeros((r, maxpos), jnp.float32)
    for j in range(m):
        counts += (pid[:, j:j + 1] == iota).astype(jnp.float32)
    possum = jnp.dot(counts, pos_ref[...], preferred_element_type=jnp.float32)
    denom = jnp.maximum(
        jnp.sum((pid != 0).astype(jnp.float32), axis=1, keepdims=True), 1.0)
    x = proj + possum / denom
    mu = jnp.mean(x, axis=-1, keepdims=True)
    xc = x - mu
    var = jnp.mean(xc * xc, axis=-1, keepdims=True)
    y = xc * lax.rsqrt(var + _EPS) * g_ref[...] + b_ref[...]
    out_ref[...] = y.reshape(bb, l, y.shape[-1])


def _tc_compute(gathered, pid, pos_table, dense_w, gamma2d, beta2d, b, l, *,
                interpret=False):
    n, emb = gathered.shape
    maxpos, hid = pos_table.shape
    m = pid.shape[1]
    bb = 32
    r = bb * l
    grid = b // bb
    return pl.pallas_call(
        functools.partial(_tc_body, maxpos, m, r, bb, l),
        grid=(grid,),
        in_specs=[
            pl.BlockSpec((r, emb), lambda i: (i, 0)),
            pl.BlockSpec((r, m), lambda i: (i, 0)),
            pl.BlockSpec((maxpos, hid), lambda i: (0, 0)),
            pl.BlockSpec((emb, hid), lambda i: (0, 0)),
            pl.BlockSpec((1, hid), lambda i: (0, 0)),
            pl.BlockSpec((1, hid), lambda i: (0, 0)),
        ],
        out_specs=pl.BlockSpec((bb, l, hid), lambda i: (i, 0, 0)),
        out_shape=jax.ShapeDtypeStruct((b, l, hid), jnp.float32),
        interpret=interpret,
    )(gathered, pid, pos_table, dense_w, gamma2d, beta2d)


def kernel(entity_ids, entity_position_ids, entity_table, pos_table, dense_w,
           ln_gamma, ln_beta):
    b, l = entity_ids.shape
    m = entity_position_ids.shape[-1]
    hid = pos_table.shape[1]
    n = b * l
    idx = entity_ids.reshape(n).astype(jnp.int32)
    gathered = _sc_gather(entity_table, idx)
    pid = entity_position_ids.reshape(n, m).astype(jnp.int32)
    return _tc_compute(gathered, pid, pos_table, dense_w,
                       ln_gamma.reshape(1, hid), ln_beta.reshape(1, hid), b, l)


# speedup vs baseline: 1.4160x; 1.4160x over previous
import functools

import jax
import jax.numpy as jnp
from jax import lax
from jax.experimental import pallas as pl
from jax.experimental.pallas import tpu as pltpu
from jax.experimental.pallas import tpu_sc as plsc

_EPS = 1e-12
_NC = 2
_NS = 16
_NW = _NC * _NS
_CHUNK = 128


def _sc_gather(table, idx):
    n = idx.shape[0]
    d = table.shape[1]
    n_chunks = n // (_NW * _CHUNK)
    idx3 = idx.reshape(_NW, n_chunks, _CHUNK)

    mesh = plsc.VectorSubcoreMesh(
        core_axis_name="c", subcore_axis_name="s",
        num_cores=_NC, num_subcores=_NS)

    @functools.partial(
        pl.kernel,
        out_type=jax.ShapeDtypeStruct((n, d), jnp.float32),
        mesh=mesh,
        scratch_types=[
            pltpu.VMEM((n_chunks, _CHUNK), jnp.int32),
            pltpu.VMEM((_CHUNK, d), jnp.float32),
            pltpu.VMEM((_CHUNK, d), jnp.float32),
            pltpu.SemaphoreType.DMA,
            pltpu.SemaphoreType.DMA,
        ],
        compiler_params=pltpu.CompilerParams(use_tc_tiling_on_sc=True),
    )
    def gather_kernel(table_hbm, idx_hbm, out_hbm, idx_v, buf0, buf1, sem0, sem1):
        wid = lax.axis_index("s") * _NC + lax.axis_index("c")
        pltpu.sync_copy(idx_hbm.at[wid], idx_v)
        bufs = (buf0, buf1)
        sems = (sem0, sem1)
        base = wid * (n_chunks * _CHUNK)
        cps = [None] * n_chunks
        cps[0] = pltpu.async_copy(table_hbm.at[idx_v.at[0]], bufs[0], sems[0])
        for j in range(n_chunks):
            if j + 1 < n_chunks:
                cps[j + 1] = pltpu.async_copy(
                    table_hbm.at[idx_v.at[j + 1]], bufs[(j + 1) % 2], sems[(j + 1) % 2])
            cps[j].wait()
            pltpu.sync_copy(bufs[j % 2], out_hbm.at[pl.ds(base + j * _CHUNK, _CHUNK)])

    return gather_kernel(table, idx3)


def _tc_body(maxpos, m, r, ent_ref, pid_ref, pos_ref, w_ref, g_ref, b_ref,
             out_ref):
    proj = jnp.dot(ent_ref[...], w_ref[...], preferred_element_type=jnp.float32)
    pid = pid_ref[...]
    iota = lax.broadcasted_iota(jnp.int32, (r, maxpos), 1)
    counts = jnp.zeros((r, maxpos), jnp.float32)
    for j in range(m):
        counts += (pid[:, j:j + 1] == iota).astype(jnp.float32)
    possum = jnp.dot(counts, pos_ref[...], preferred_element_type=jnp.float32)
    denom = jnp.maximum(
        jnp.sum((pid != 0).astype(jnp.float32), axis=1, keepdims=True), 1.0)
    x = proj + possum / denom
    mu = jnp.mean(x, axis=-1, keepdims=True)
    xc = x - mu
    var = jnp.mean(xc * xc, axis=-1, keepdims=True)
    y = xc * lax.rsqrt(var + _EPS) * g_ref[...] + b_ref[...]
    out_ref[...] = y.reshape(1, r, y.shape[-1])


def _tc_compute(gathered, pid, pos_table, dense_w, gamma2d, beta2d, b, l, *,
                interpret=False):
    n, emb = gathered.shape
    maxpos, hid = pos_table.shape
    m = pid.shape[1]
    r = b
    grid = l
    return pl.pallas_call(
        functools.partial(_tc_body, maxpos, m, r),
        grid=(grid,),
        in_specs=[
            pl.BlockSpec((r, emb), lambda i: (i, 0)),
            pl.BlockSpec((r, m), lambda i: (i, 0)),
            pl.BlockSpec((maxpos, hid), lambda i: (0, 0)),
            pl.BlockSpec((emb, hid), lambda i: (0, 0)),
            pl.BlockSpec((1, hid), lambda i: (0, 0)),
            pl.BlockSpec((1, hid), lambda i: (0, 0)),
        ],
        out_specs=pl.BlockSpec((1, r, hid), lambda i: (i, 0, 0)),
        out_shape=jax.ShapeDtypeStruct((l, b, hid), jnp.float32),
        interpret=interpret,
    )(gathered, pid, pos_table, dense_w, gamma2d, beta2d)


def kernel(entity_ids, entity_position_ids, entity_table, pos_table, dense_w,
           ln_gamma, ln_beta):
    b, l = entity_ids.shape
    m = entity_position_ids.shape[-1]
    hid = pos_table.shape[1]
    n = b * l
    idx = entity_ids.T.reshape(n).astype(jnp.int32)
    gathered = _sc_gather(entity_table, idx)
    pid = entity_position_ids.transpose(1, 0, 2).reshape(n, m).astype(jnp.int32)
    out = _tc_compute(gathered, pid, pos_table, dense_w,
                      ln_gamma.reshape(1, hid), ln_beta.reshape(1, hid), b, l)
    return out.transpose(1, 0, 2)
